# baseline (speedup 1.0000x reference)
import jax
import jax.numpy as jnp
from jax.experimental import pallas as pl
from jax.experimental.pallas import tpu as pltpu

B, T, D, H = 8, 2048, 256, 256
CHUNK = 256
NCHUNK = T // CHUNK


def _gru2_kernel(xt_ref, wih0_ref, whh0_ref, wih1_ref, whh1_ref,
                 cb0_ref, cb1_ref, bn0_ref, bn1_ref,
                 out_ref, gi1_ref, gi2_ref, h1buf_ref, h1_ref, h2_ref):
    c = pl.program_id(0)

    @pl.when(c == 0)
    def _init():
        h1_ref[...] = jnp.zeros_like(h1_ref)
        h2_ref[...] = jnp.zeros_like(h2_ref)

    @pl.when(c >= 1)
    def _gi2():
        gi2_ref[...] = (jnp.dot(h1buf_ref[...], wih1_ref[...],
                                preferred_element_type=jnp.float32)
                        + cb1_ref[...])

    @pl.when(c < NCHUNK)
    def _gi1():
        gi1_ref[...] = (jnp.dot(xt_ref[...], wih0_ref[...],
                                preferred_element_type=jnp.float32)
                        + cb0_ref[...])

    def cell(gi, gh, bn, h):
        rz = jax.nn.sigmoid(gi[:, :2 * H] + gh[:, :2 * H])
        n = jnp.tanh(gi[:, 2 * H:] + rz[:, :H] * (gh[:, 2 * H:] + bn))
        return n + rz[:, H:] * (h - n)

    def sub1(t, h1):
        gh1 = jnp.dot(h1.astype(jnp.bfloat16), whh0_ref[...],
                      preferred_element_type=jnp.float32)
        h1 = cell(gi1_ref[pl.ds(t * B, B), :], gh1, bn0_ref[...], h1)
        h1buf_ref[pl.ds(t * B, B), :] = h1
        return h1

    def sub2(t, h2):
        gh2 = jnp.dot(h2.astype(jnp.bfloat16), whh1_ref[...],
                      preferred_element_type=jnp.float32)
        h2 = cell(gi2_ref[pl.ds(t * B, B), :], gh2, bn1_ref[...], h2)
        out_ref[pl.ds(t * B, B), :] = h2
        return h2

    @pl.when(c == 0)
    def _first():
        def step(i, h1):
            for k in range(16):
                h1 = sub1(16 * i + k, h1)
            return h1
        h1_ref[...] = jax.lax.fori_loop(0, CHUNK // 16, step, h1_ref[...])

    @pl.when(jnp.logical_and(c >= 1, c < NCHUNK))
    def _main():
        def step(i, carry):
            h1, h2 = carry
            for k in range(0, 16, 2):
                t = 16 * i + k
                h1 = sub1(t, h1)
                h1 = sub1(t + 1, h1)
                h2 = sub2(t, h2)
                h2 = sub2(t + 1, h2)
            return (h1, h2)
        h1, h2 = jax.lax.fori_loop(0, CHUNK // 16, step,
                                   (h1_ref[...], h2_ref[...]))
        h1_ref[...] = h1
        h2_ref[...] = h2

    @pl.when(c == NCHUNK)
    def _drain():
        def step(i, h2):
            for k in range(16):
                h2 = sub2(16 * i + k, h2)
            return h2
        h2_ref[...] = jax.lax.fori_loop(0, CHUNK // 16, step, h2_ref[...])


def kernel(x, w_ih_l0, w_hh_l0, b_ih_l0, b_hh_l0,
           w_ih_l1, w_hh_l1, b_ih_l1, b_hh_l1):
    xt = jnp.swapaxes(x, 0, 1).reshape(T * B, D)

    cb0 = jnp.concatenate([(b_ih_l0[:2 * H] + b_hh_l0[:2 * H]),
                           b_ih_l0[2 * H:]]).reshape(1, -1)
    cb1 = jnp.concatenate([(b_ih_l1[:2 * H] + b_hh_l1[:2 * H]),
                           b_ih_l1[2 * H:]]).reshape(1, -1)
    bn0 = b_hh_l0[2 * H:].reshape(1, -1)
    bn1 = b_hh_l1[2 * H:].reshape(1, -1)

    full2d = lambda shape: pl.BlockSpec(shape, lambda i: (0, 0))
    out2d = pl.pallas_call(
        _gru2_kernel,
        grid=(NCHUNK + 1,),
        in_specs=[
            pl.BlockSpec((CHUNK * B, D),
                         lambda c: (jnp.minimum(c, NCHUNK - 1), 0)),
            full2d((D, 3 * H)),
            full2d((H, 3 * H)),
            full2d((H, 3 * H)),
            full2d((H, 3 * H)),
            full2d((1, 3 * H)),
            full2d((1, 3 * H)),
            full2d((1, H)),
            full2d((1, H)),
        ],
        out_specs=pl.BlockSpec((CHUNK * B, H),
                               lambda c: (jnp.maximum(c - 1, 0), 0)),
        out_shape=jax.ShapeDtypeStruct((T * B, H), jnp.float32),
        scratch_shapes=[
            pltpu.VMEM((CHUNK * B, 3 * H), jnp.float32),
            pltpu.VMEM((CHUNK * B, 3 * H), jnp.float32),
            pltpu.VMEM((CHUNK * B, H), jnp.float32),
            pltpu.VMEM((B, H), jnp.float32),
            pltpu.VMEM((B, H), jnp.float32),
        ],
        compiler_params=pltpu.CompilerParams(
            dimension_semantics=("arbitrary",),
        ),
    )(xt, w_ih_l0.T, w_hh_l0.T.astype(jnp.bfloat16),
      w_ih_l1.T, w_hh_l1.T.astype(jnp.bfloat16),
      cb0, cb1, bn0, bn1)

    return jnp.swapaxes(out2d.reshape(T, B, H), 0, 1)


# speedup vs baseline: 1.1822x; 1.1822x over previous
import jax
import jax.numpy as jnp
from jax.experimental import pallas as pl
from jax.experimental.pallas import tpu as pltpu

B, T, D, H = 8, 2048, 256, 256
CHUNK = 256
NCHUNK = T // CHUNK


def _gru2_kernel(xt_ref, wih0_ref, whh0_ref, wih1_ref, whh1_ref,
                 cb0_ref, cb1_ref, bn0_ref, bn1_ref,
                 out_ref, gi1_ref, gi2_ref, h1buf_ref, h1_ref, h2_ref):
    c = pl.program_id(0)

    @pl.when(c == 0)
    def _init():
        h1_ref[...] = jnp.zeros_like(h1_ref)
        h2_ref[...] = jnp.zeros_like(h2_ref)

    @pl.when(c >= 1)
    def _gi2():
        gi2_ref[...] = (jnp.dot(h1buf_ref[...], wih1_ref[...],
                                preferred_element_type=jnp.float32)
                        + cb1_ref[...])

    @pl.when(c < NCHUNK)
    def _gi1():
        gi1_ref[...] = (jnp.dot(xt_ref[...], wih0_ref[...],
                                preferred_element_type=jnp.float32)
                        + cb0_ref[...])

    def cell(gi, gh, bn, h):
        rz = jax.nn.sigmoid(gi[:, :2 * H] + gh[:, :2 * H])
        n = jnp.tanh(gi[:, 2 * H:] + rz[:, :H] * (gh[:, 2 * H:] + bn))
        return n + rz[:, H:] * (h - n)

    def sub1(t, h1):
        gh1 = jnp.dot(h1.astype(jnp.bfloat16), whh0_ref[...],
                      preferred_element_type=jnp.float32)
        h1 = cell(gi1_ref[pl.ds(t * B, B), :], gh1, bn0_ref[...], h1)
        h1buf_ref[pl.ds(t * B, B), :] = h1
        return h1

    def sub2(t, h2):
        gh2 = jnp.dot(h2.astype(jnp.bfloat16), whh1_ref[...],
                      preferred_element_type=jnp.float32)
        h2 = cell(gi2_ref[pl.ds(t * B, B), :], gh2, bn1_ref[...], h2)
        out_ref[pl.ds(t * B, B), :] = h2
        return h2

    @pl.when(c == 0)
    def _first():
        def step(i, h1):
            for k in range(16):
                h1 = sub1(16 * i + k, h1)
            return h1
        h1_ref[...] = jax.lax.fori_loop(0, CHUNK // 16, step, h1_ref[...])

    @pl.when(jnp.logical_and(c >= 1, c < NCHUNK))
    def _main():
        def step(i, carry):
            h1, h2 = carry
            for k in range(16):
                t = 16 * i + k
                h1 = sub1(t, h1)
                h2 = sub2(t, h2)
            return (h1, h2)
        h1, h2 = jax.lax.fori_loop(0, CHUNK // 16, step,
                                   (h1_ref[...], h2_ref[...]))
        h1_ref[...] = h1
        h2_ref[...] = h2

    @pl.when(c == NCHUNK)
    def _drain():
        def step(i, h2):
            for k in range(16):
                h2 = sub2(16 * i + k, h2)
            return h2
        h2_ref[...] = jax.lax.fori_loop(0, CHUNK // 16, step, h2_ref[...])


def kernel(x, w_ih_l0, w_hh_l0, b_ih_l0, b_hh_l0,
           w_ih_l1, w_hh_l1, b_ih_l1, b_hh_l1):
    xt = jnp.swapaxes(x, 0, 1).reshape(T * B, D)

    cb0 = jnp.concatenate([(b_ih_l0[:2 * H] + b_hh_l0[:2 * H]),
                           b_ih_l0[2 * H:]]).reshape(1, -1)
    cb1 = jnp.concatenate([(b_ih_l1[:2 * H] + b_hh_l1[:2 * H]),
                           b_ih_l1[2 * H:]]).reshape(1, -1)
    bn0 = b_hh_l0[2 * H:].reshape(1, -1)
    bn1 = b_hh_l1[2 * H:].reshape(1, -1)

    full2d = lambda shape: pl.BlockSpec(shape, lambda i: (0, 0))
    out2d = pl.pallas_call(
        _gru2_kernel,
        grid=(NCHUNK + 1,),
        in_specs=[
            pl.BlockSpec((CHUNK * B, D),
                         lambda c: (jnp.minimum(c, NCHUNK - 1), 0)),
            full2d((D, 3 * H)),
            full2d((H, 3 * H)),
            full2d((H, 3 * H)),
            full2d((H, 3 * H)),
            full2d((1, 3 * H)),
            full2d((1, 3 * H)),
            full2d((1, H)),
            full2d((1, H)),
        ],
        out_specs=pl.BlockSpec((CHUNK * B, H),
                               lambda c: (jnp.maximum(c - 1, 0), 0)),
        out_shape=jax.ShapeDtypeStruct((T * B, H), jnp.float32),
        scratch_shapes=[
            pltpu.VMEM((CHUNK * B, 3 * H), jnp.float32),
            pltpu.VMEM((CHUNK * B, 3 * H), jnp.float32),
            pltpu.VMEM((CHUNK * B, H), jnp.float32),
            pltpu.VMEM((B, H), jnp.float32),
            pltpu.VMEM((B, H), jnp.float32),
        ],
        compiler_params=pltpu.CompilerParams(
            dimension_semantics=("arbitrary",),
        ),
    )(xt, w_ih_l0.T, w_hh_l0.T.astype(jnp.bfloat16),
      w_ih_l1.T, w_hh_l1.T.astype(jnp.bfloat16),
      cb0, cb1, bn0, bn1)

    return jnp.swapaxes(out2d.reshape(T, B, H), 0, 1)
